# baseline (speedup 1.0000x reference)
import jax
import jax.numpy as jnp
from jax.experimental import pallas as pl


def _bcast_body(emb_ref, out_ref):
    out_ref[...] = jnp.broadcast_to(emb_ref[...], out_ref.shape)


def kernel(inputs, embedding):
    batch = inputs.shape[0]
    d = embedding.shape[1]
    tile = 2048
    return pl.pallas_call(
        _bcast_body,
        grid=(batch // tile,),
        in_specs=[pl.BlockSpec((1, d), lambda i: (0, 0))],
        out_specs=pl.BlockSpec((tile, d), lambda i: (i, 0)),
        out_shape=jax.ShapeDtypeStruct((batch, d), embedding.dtype),
    )(embedding)
